# baseline (speedup 1.0000x reference)
import dataclasses
import functools

import jax
import jax.numpy as jnp
from jax import lax
from jax.experimental import pallas as pl
from jax.experimental.pallas import tpu as pltpu
from jax.experimental.pallas import tpu_sc as plsc

FEAT = 128
HEADS = 4
MAX_GATE = 99
N_PER_GRAPH = MAX_GATE + 1
NUM_GRAPHS = 100
D = 2 * FEAT

N = 10000
E2 = 160000 + N
NW = 32
NP = 10240
GP = 104
SRC_PAD = E2 + 4096
DEG_PAD = N + 256
WTAB = 96


def _sc_params():
    cp = pltpu.CompilerParams()
    if "needs_layout_passes" in pltpu.CompilerParams.__dataclass_fields__:
        cp = dataclasses.replace(cp, needs_layout_passes=False)
    return cp


def _iota16():
    return lax.iota(jnp.int32, 16)


def _extract_i32(ref, pos, base, nchunks):
    acc = jnp.int32(0)
    for k in range(nchunks):
        v = ref[pl.ds(base + k * 16, 16)]
        acc = acc + jnp.sum(jnp.where(_iota16() + (base + k * 16) == pos, v, 0))
    return acc


def _extract_dyn_i32(ref, pos):
    g = (pos // 16) * 16
    v = ref[pl.ds(g, 16)]
    return jnp.sum(jnp.where(_iota16() == pos - g, v, 0))


def _emb_gather(emb, idxp):
    B = idxp.shape[0]
    b_per_w = B // NW
    mesh = plsc.VectorSubcoreMesh(core_axis_name="c", subcore_axis_name="s")

    @functools.partial(
        pl.kernel,
        out_type=jax.ShapeDtypeStruct((B, FEAT), jnp.float32),
        mesh=mesh,
        compiler_params=_sc_params(),
        scratch_types=[
            pltpu.VMEM((b_per_w,), jnp.int32),
            pltpu.VMEM((b_per_w, FEAT), jnp.float32),
            pltpu.SemaphoreType.DMA,
        ],
    )
    def k(emb_hbm, idx_hbm, out_hbm, idx_v, rows_v, sem):
        w = lax.axis_index("c") * 16 + lax.axis_index("s")
        base = w * b_per_w
        pltpu.sync_copy(idx_hbm.at[pl.ds(base, b_per_w)], idx_v)
        pltpu.async_copy(emb_hbm.at[idx_v], rows_v, sem).wait()
        pltpu.sync_copy(rows_v, out_hbm.at[pl.ds(base, b_per_w)])

    return k(emb, idxp)


def _matmul_bias(xp, w, b):
    Nr, K = xp.shape
    M = w.shape[1]
    BM, BN = 400, min(M, 512)
    grid = (Nr // BM, M // BN)

    def mk(x_ref, w_ref, b_ref, o_ref):
        o_ref[...] = (
            jnp.dot(x_ref[...], w_ref[...], preferred_element_type=jnp.float32)
            + b_ref[...]
        )

    return pl.pallas_call(
        mk,
        grid=grid,
        in_specs=[
            pl.BlockSpec((BM, K), lambda i, j: (i, 0)),
            pl.BlockSpec((K, BN), lambda i, j: (0, j)),
            pl.BlockSpec((1, BN), lambda i, j: (0, j)),
        ],
        out_specs=pl.BlockSpec((BM, BN), lambda i, j: (i, j)),
        out_shape=jax.ShapeDtypeStruct((Nr, M), jnp.float32),
    )(xp, w, b[None, :])


def _edge_layer(xl, xr, srcp, degp, wtab, attf, bof, heads):
    HC = heads * 256
    mesh = plsc.VectorSubcoreMesh(core_axis_name="c", subcore_axis_name="s")

    @functools.partial(
        pl.kernel,
        out_type=jax.ShapeDtypeStruct((N, HC), jnp.float32),
        mesh=mesh,
        compiler_params=_sc_params(),
        scratch_types=[
            pltpu.VMEM((2048,), jnp.int32),
            pltpu.VMEM((256,), jnp.int32),
            pltpu.VMEM((WTAB,), jnp.int32),
            pltpu.VMEM((16, HC), jnp.float32),
            pltpu.VMEM((16, HC), jnp.float32),
            pltpu.VMEM((1, HC), jnp.float32),
            pltpu.VMEM((1, HC), jnp.float32),
            pltpu.VMEM((1, HC), jnp.float32),
            pltpu.VMEM((1, HC), jnp.float32),
            pltpu.VMEM((1, HC), jnp.float32),
            pltpu.SemaphoreType.DMA,
            pltpu.SemaphoreType.DMA,
            pltpu.SemaphoreType.DMA,
        ],
    )
    def k(xl_hbm, xr_hbm, src_hbm, deg_hbm, wtab_hbm, att_hbm, bo_hbm,
          out_hbm, srcbuf, degbuf, wtabbuf, rowsA, rowsB, xrbuf, accbuf,
          outbuf, attbuf, bobuf, sem, semA, semB):
        w = lax.axis_index("c") * 16 + lax.axis_index("s")
        pltpu.sync_copy(wtab_hbm, wtabbuf)
        pltpu.sync_copy(att_hbm, attbuf)
        pltpu.sync_copy(bo_hbm, bobuf)
        e0 = _extract_i32(wtabbuf, w, 0, 3)
        n0 = _extract_i32(wtabbuf, 48 + w, 48, 3)
        n1 = _extract_i32(wtabbuf, 48 + w + 1, 48, 3)

        def prep_issue(epos, cb_, rowsX, semX):
            need_src = jnp.logical_or(epos + 16 > cb_ + 2048, cb_ < 0)

            def refill_src():
                sb_ = (epos // 16) * 16
                pltpu.sync_copy(src_hbm.at[pl.ds(sb_, 2048)], srcbuf)
                return sb_

            cb_ = lax.cond(need_src, refill_src, lambda: cb_)
            idxv = srcbuf[pl.ds(epos - cb_, 16)]
            pltpu.make_async_copy(xl_hbm.at[idxv], rowsX, semX).start()
            return cb_

        def wait_rows(rowsX, semX):
            dummy = jnp.zeros((16,), jnp.int32)
            pltpu.make_async_copy(xl_hbm.at[dummy], rowsX, semX).wait()

        def compute_win(wk, rowsX, st, d):
            drem = d - wk * 16
            lanemask = _iota16() < drem
            newst = []
            for h in range(heads):
                def aq(q, accs, h=h):
                    c0 = h * 256 + q * 16
                    xrv = xrbuf[0, pl.ds(c0, 16)]
                    atv = attbuf[0, pl.ds(c0, 16)]
                    out = []
                    for j in range(16):
                        z = rowsX[j, pl.ds(c0, 16)] + xrv
                        z = jnp.maximum(z, 0.2 * z)
                        out.append(accs[j] + z * atv)
                    return tuple(out)

                accs = lax.fori_loop(
                    0, 16, aq,
                    tuple(jnp.zeros((16,), jnp.float32) for _ in range(16)),
                )
                alph = jnp.full((16,), -jnp.inf, jnp.float32)
                for j in range(16):
                    alph = jnp.where(_iota16() == j, jnp.sum(accs[j]), alph)
                alph = jnp.where(lanemask, alph, -jnp.inf)
                m_old, den_old = st[h]
                m_new = jnp.maximum(m_old, jnp.max(alph))
                rv = jnp.exp(jnp.full((16,), m_old - m_new, jnp.float32))
                r_s = jnp.max(rv)
                wv = jnp.exp(alph - m_new)
                den_new = den_old * r_s + jnp.sum(wv)
                wsp = [wv[jnp.full((16,), j, jnp.int32)] for j in range(16)]

                def wq(q, _, h=h, wsp=wsp, r_s=r_s):
                    c0 = h * 256 + q * 16
                    a_ = accbuf[0, pl.ds(c0, 16)] * r_s
                    for j in range(16):
                        a_ = a_ + wsp[j] * rowsX[j, pl.ds(c0, 16)]
                    accbuf[0, pl.ds(c0, 16)] = a_
                    return 0

                lax.fori_loop(0, 16, wq, 0)
                newst.append((m_new, den_new))
            return tuple(newst)

        def node_body(n, carry):
            e, cb, db = carry

            need_deg = jnp.logical_or(n >= db + 256, db < 0)

            def refill_deg():
                nb_ = (n // 16) * 16
                pltpu.sync_copy(deg_hbm.at[pl.ds(nb_, 256)], degbuf)
                return nb_

            db = lax.cond(need_deg, refill_deg, lambda: db)
            d = _extract_dyn_i32(degbuf, n - db)

            xr_cp = pltpu.make_async_copy(xr_hbm.at[pl.ds(n, 1)], xrbuf, sem)
            xr_cp.start()

            nwin = (d + 15) // 16
            cb = lax.cond(nwin > 0,
                          lambda: prep_issue(e, cb, rowsA, semA),
                          lambda: cb)

            def zero_q(q, _):
                accbuf[0, pl.ds(q * 16, 16)] = jnp.zeros((16,), jnp.float32)
                return 0

            lax.fori_loop(0, HC // 16, zero_q, 0)
            xr_cp.wait()

            st0 = tuple(
                (jnp.float32(-jnp.inf), jnp.float32(0.0)) for _ in range(heads)
            )
            npair = (nwin + 1) // 2

            def pair_body(t, wc):
                eP, cbP, st = wc
                wait_rows(rowsA, semA)
                validB = 2 * t + 1 < nwin
                cbP = lax.cond(validB,
                               lambda: prep_issue(eP + 16, cbP, rowsB, semB),
                               lambda: cbP)
                st = compute_win(2 * t, rowsA, st, d)
                validA2 = 2 * t + 2 < nwin
                cbP = lax.cond(validA2,
                               lambda: prep_issue(eP + 32, cbP, rowsA, semA),
                               lambda: cbP)

                def do_b():
                    wait_rows(rowsB, semB)
                    return compute_win(2 * t + 1, rowsB, st, d)

                st = lax.cond(validB, do_b, lambda: st)
                return eP + 32, cbP, st

            _, cb, stF = lax.fori_loop(0, npair, pair_body, (e, cb, st0))

            for h in range(heads):
                _, den_h = stF[h]
                invv = 1.0 / (jnp.full((16,), den_h, jnp.float32) + 1e-16)

                def fq(q, _, h=h, invv=invv):
                    c0 = h * 256 + q * 16
                    o = accbuf[0, pl.ds(c0, 16)] * invv + bobuf[0, pl.ds(c0, 16)]
                    outbuf[0, pl.ds(c0, 16)] = jnp.maximum(o, 0.01 * o)
                    return 0

                lax.fori_loop(0, 16, fq, 0)

            pltpu.sync_copy(outbuf, out_hbm.at[pl.ds(n, 1)])
            return e + d, cb, db

        lax.fori_loop(
            n0, n1, node_body,
            (e0, jnp.int32(-(2 ** 30)), jnp.int32(-(2 ** 30))),
        )

    return k(xl, xr, srcp, degp, wtab, attf, bof)


def _final_ln(s4p, xf0p, batch_row, batch_col, lnw, lnb):
    def fk(s_ref, x0_ref, b_ref, bc_ref, w_ref, bb_ref, o_ref):
        h = s_ref[...] + x0_ref[...]
        bt = b_ref[...]
        gi = lax.broadcasted_iota(jnp.int32, (GP, NP), 0)
        oh = jnp.where(gi == bt, 1.0, 0.0).astype(jnp.float32)
        ni = lax.broadcasted_iota(jnp.int32, (GP, NP), 1)
        psel = jnp.where(ni == gi * 100, 1.0, 0.0).astype(jnp.float32)

        rs = jnp.sum(h, axis=1, keepdims=True)
        rq = jnp.sum(h * h, axis=1, keepdims=True)
        ones = jnp.ones((NP, 1), jnp.float32)
        stats = jnp.concatenate([rs, rq, ones], axis=1)
        g = jnp.dot(oh, stats, preferred_element_type=jnp.float32)
        cnt = g[:, 2:3]
        norm = jnp.clip(cnt, 1.0, None) * 256.0
        mean = g[:, 0:1] / norm
        var = g[:, 1:2] / norm - mean * mean
        rstd = lax.rsqrt(var + 1e-5)

        h_sel = jnp.dot(psel, h, preferred_element_type=jnp.float32)
        x0_sel = jnp.dot(psel, x0_ref[...], preferred_element_type=jnp.float32)
        b_sel = jnp.dot(psel, bc_ref[...],
                        preferred_element_type=jnp.float32)
        gi2 = lax.broadcasted_iota(jnp.int32, (GP, GP), 1)
        q = jnp.where(b_sel == gi2.astype(jnp.float32), 1.0, 0.0)
        mean_sel = jnp.dot(q, mean, preferred_element_type=jnp.float32)
        rstd_sel = jnp.dot(q, rstd, preferred_element_type=jnp.float32)

        o_ref[...] = ((h_sel - mean_sel) * rstd_sel * w_ref[...] + bb_ref[...]
                      + x0_sel)

    return pl.pallas_call(
        fk,
        out_shape=jax.ShapeDtypeStruct((GP, D), jnp.float32),
    )(s4p, xf0p, batch_row, batch_col, lnw[None, :], lnb[None, :])


def kernel(x, edge_index, batch, params):
    loops = jnp.arange(N, dtype=edge_index.dtype)
    src = jnp.concatenate([edge_index[0], loops])
    dst = jnp.concatenate([edge_index[1], loops])

    perm = jnp.argsort(dst)
    dst_s = dst[perm]
    src_s = src[perm].astype(jnp.int32)
    noff = jnp.searchsorted(dst_s, jnp.arange(N + 1)).astype(jnp.int32)
    deg = noff[1:] - noff[:-1]
    degp = jnp.pad(deg, (0, DEG_PAD - N))
    t = (jnp.arange(33) * E2) // 32
    nb = jnp.searchsorted(noff, t).astype(jnp.int32)
    eoff = noff[nb]
    wtab = jnp.zeros((WTAB,), jnp.int32)
    wtab = wtab.at[0:33].set(eoff).at[48:81].set(nb)
    perm2 = jnp.argsort(src)
    srcp = jnp.pad(src_s, (0, SRC_PAD - E2)) + jnp.minimum(perm2[0], 0)

    idxp = jnp.pad(x.reshape(-1).astype(jnp.int32), (0, 20480 - 2 * N))
    xf = _emb_gather(params["emb"], idxp)[: 2 * N].reshape(N, D)
    res = xf

    L = params["layers"]
    for i in range(5):
        p = L[i]
        heads = HEADS if i < 4 else 1
        xl = _matmul_bias(xf, p["Wl"], p["bl"])
        xr = _matmul_bias(xf, p["Wr"], p["br"])
        attf = p["att"].reshape(1, -1)
        bof = p["bo"][None, :]
        xf = _edge_layer(xl, xr, srcp, degp, wtab, attf, bof, heads)

    s4p = jnp.pad(xf, ((0, NP - N), (0, 0)))
    xf0p = jnp.pad(res, ((0, NP - N), (0, 0)))
    batchp = jnp.pad(batch.astype(jnp.int32), (0, NP - N),
                     constant_values=NUM_GRAPHS + 7)
    batch_row = batchp[None, :]
    batch_col = jnp.where(batchp < NUM_GRAPHS, batchp, 0).astype(jnp.float32)[:, None]
    out = _final_ln(s4p, xf0p, batch_row, batch_col,
                    params["ln_w"], params["ln_b"])
    return out[:NUM_GRAPHS]


# speedup vs baseline: 1.0112x; 1.0112x over previous
import dataclasses
import functools

import jax
import jax.numpy as jnp
from jax import lax
from jax.experimental import pallas as pl
from jax.experimental.pallas import tpu as pltpu
from jax.experimental.pallas import tpu_sc as plsc

FEAT = 128
HEADS = 4
MAX_GATE = 99
N_PER_GRAPH = MAX_GATE + 1
NUM_GRAPHS = 100
D = 2 * FEAT

N = 10000
E2 = 160000 + N
NW = 32
NP = 10240
GP = 104
SRC_PAD = E2 + 4096
DEG_PAD = N + 256
WTAB = 96


def _sc_params():
    cp = pltpu.CompilerParams()
    if "needs_layout_passes" in pltpu.CompilerParams.__dataclass_fields__:
        cp = dataclasses.replace(cp, needs_layout_passes=False)
    return cp


def _iota16():
    return lax.iota(jnp.int32, 16)


def _extract_i32(ref, pos, base, nchunks):
    acc = jnp.int32(0)
    for k in range(nchunks):
        v = ref[pl.ds(base + k * 16, 16)]
        acc = acc + jnp.sum(jnp.where(_iota16() + (base + k * 16) == pos, v, 0))
    return acc


def _extract_dyn_i32(ref, pos):
    g = (pos // 16) * 16
    v = ref[pl.ds(g, 16)]
    return jnp.sum(jnp.where(_iota16() == pos - g, v, 0))


def _emb_gather(emb, idxp):
    B = idxp.shape[0]
    b_per_w = B // NW
    mesh = plsc.VectorSubcoreMesh(core_axis_name="c", subcore_axis_name="s")

    @functools.partial(
        pl.kernel,
        out_type=jax.ShapeDtypeStruct((B, FEAT), jnp.float32),
        mesh=mesh,
        compiler_params=_sc_params(),
        scratch_types=[
            pltpu.VMEM((b_per_w,), jnp.int32),
            pltpu.VMEM((b_per_w, FEAT), jnp.float32),
            pltpu.SemaphoreType.DMA,
        ],
    )
    def k(emb_hbm, idx_hbm, out_hbm, idx_v, rows_v, sem):
        w = lax.axis_index("c") * 16 + lax.axis_index("s")
        base = w * b_per_w
        pltpu.sync_copy(idx_hbm.at[pl.ds(base, b_per_w)], idx_v)
        pltpu.async_copy(emb_hbm.at[idx_v], rows_v, sem).wait()
        pltpu.sync_copy(rows_v, out_hbm.at[pl.ds(base, b_per_w)])

    return k(emb, idxp)


def _matmul_bias(xp, w, b):
    Nr, K = xp.shape
    M = w.shape[1]
    BM, BN = 400, min(M, 512)
    grid = (Nr // BM, M // BN)

    def mk(x_ref, w_ref, b_ref, o_ref):
        o_ref[...] = (
            jnp.dot(x_ref[...].astype(jnp.bfloat16),
                    w_ref[...].astype(jnp.bfloat16),
                    preferred_element_type=jnp.float32)
            + b_ref[...]
        )

    return pl.pallas_call(
        mk,
        grid=grid,
        in_specs=[
            pl.BlockSpec((BM, K), lambda i, j: (i, 0)),
            pl.BlockSpec((K, BN), lambda i, j: (0, j)),
            pl.BlockSpec((1, BN), lambda i, j: (0, j)),
        ],
        out_specs=pl.BlockSpec((BM, BN), lambda i, j: (i, j)),
        out_shape=jax.ShapeDtypeStruct((Nr, M), jnp.float32),
    )(xp, w, b[None, :])


def _edge_layer(xl, xr, srcp, degp, wtab, attf, bof, heads):
    HC = heads * 256
    mesh = plsc.VectorSubcoreMesh(core_axis_name="c", subcore_axis_name="s")

    @functools.partial(
        pl.kernel,
        out_type=jax.ShapeDtypeStruct((N, HC), jnp.float32),
        mesh=mesh,
        compiler_params=_sc_params(),
        scratch_types=[
            pltpu.VMEM((2048,), jnp.int32),
            pltpu.VMEM((256,), jnp.int32),
            pltpu.VMEM((WTAB,), jnp.int32),
            pltpu.VMEM((16, HC), jnp.float32),
            pltpu.VMEM((16, HC), jnp.float32),
            pltpu.VMEM((1, HC), jnp.float32),
            pltpu.VMEM((1, HC), jnp.float32),
            pltpu.VMEM((1, HC), jnp.float32),
            pltpu.VMEM((1, HC), jnp.float32),
            pltpu.VMEM((1, HC), jnp.float32),
            pltpu.SemaphoreType.DMA,
            pltpu.SemaphoreType.DMA,
            pltpu.SemaphoreType.DMA,
        ],
    )
    def k(xl_hbm, xr_hbm, src_hbm, deg_hbm, wtab_hbm, att_hbm, bo_hbm,
          out_hbm, srcbuf, degbuf, wtabbuf, rowsA, rowsB, xrbuf, accbuf,
          outbuf, attbuf, bobuf, sem, semA, semB):
        w = lax.axis_index("c") * 16 + lax.axis_index("s")
        pltpu.sync_copy(wtab_hbm, wtabbuf)
        pltpu.sync_copy(att_hbm, attbuf)
        pltpu.sync_copy(bo_hbm, bobuf)
        e0 = _extract_i32(wtabbuf, w, 0, 3)
        n0 = _extract_i32(wtabbuf, 48 + w, 48, 3)
        n1 = _extract_i32(wtabbuf, 48 + w + 1, 48, 3)

        def prep_issue(epos, cb_, rowsX, semX):
            need_src = jnp.logical_or(epos + 16 > cb_ + 2048, cb_ < 0)

            def refill_src():
                sb_ = (epos // 16) * 16
                pltpu.sync_copy(src_hbm.at[pl.ds(sb_, 2048)], srcbuf)
                return sb_

            cb_ = lax.cond(need_src, refill_src, lambda: cb_)
            idxv = srcbuf[pl.ds(epos - cb_, 16)]
            pltpu.make_async_copy(xl_hbm.at[idxv], rowsX, semX).start()
            return cb_

        def wait_rows(rowsX, semX):
            dummy = jnp.zeros((16,), jnp.int32)
            pltpu.make_async_copy(xl_hbm.at[dummy], rowsX, semX).wait()

        def compute_win(wk, rowsX, st, d):
            drem = d - wk * 16
            lanemask = _iota16() < drem
            newst = []
            for h in range(heads):
                def aq(q, accs, h=h):
                    c0 = h * 256 + q * 16
                    xrv = xrbuf[0, pl.ds(c0, 16)]
                    atv = attbuf[0, pl.ds(c0, 16)]
                    out = []
                    for j in range(16):
                        z = rowsX[j, pl.ds(c0, 16)] + xrv
                        z = jnp.maximum(z, 0.2 * z)
                        out.append(accs[j] + z * atv)
                    return tuple(out)

                accs = lax.fori_loop(
                    0, 16, aq,
                    tuple(jnp.zeros((16,), jnp.float32) for _ in range(16)),
                )
                alph = jnp.full((16,), -jnp.inf, jnp.float32)
                for j in range(16):
                    alph = jnp.where(_iota16() == j, jnp.sum(accs[j]), alph)
                alph = jnp.where(lanemask, alph, -jnp.inf)
                m_old, den_old = st[h]
                m_new = jnp.maximum(m_old, jnp.max(alph))
                rv = jnp.exp(jnp.full((16,), m_old - m_new, jnp.float32))
                r_s = jnp.max(rv)
                wv = jnp.exp(alph - m_new)
                den_new = den_old * r_s + jnp.sum(wv)
                wsp = [wv[jnp.full((16,), j, jnp.int32)] for j in range(16)]

                def wq(q, _, h=h, wsp=wsp, r_s=r_s):
                    c0 = h * 256 + q * 16
                    a_ = accbuf[0, pl.ds(c0, 16)] * r_s
                    for j in range(16):
                        a_ = a_ + wsp[j] * rowsX[j, pl.ds(c0, 16)]
                    accbuf[0, pl.ds(c0, 16)] = a_
                    return 0

                lax.fori_loop(0, 16, wq, 0)
                newst.append((m_new, den_new))
            return tuple(newst)

        def node_body(n, carry):
            e, cb, db = carry

            need_deg = jnp.logical_or(n >= db + 256, db < 0)

            def refill_deg():
                nb_ = (n // 16) * 16
                pltpu.sync_copy(deg_hbm.at[pl.ds(nb_, 256)], degbuf)
                return nb_

            db = lax.cond(need_deg, refill_deg, lambda: db)
            d = _extract_dyn_i32(degbuf, n - db)

            xr_cp = pltpu.make_async_copy(xr_hbm.at[pl.ds(n, 1)], xrbuf, sem)
            xr_cp.start()

            nwin = (d + 15) // 16
            cb = lax.cond(nwin > 0,
                          lambda: prep_issue(e, cb, rowsA, semA),
                          lambda: cb)

            def zero_q(q, _):
                accbuf[0, pl.ds(q * 16, 16)] = jnp.zeros((16,), jnp.float32)
                return 0

            lax.fori_loop(0, HC // 16, zero_q, 0)
            xr_cp.wait()

            st0 = tuple(
                (jnp.float32(-jnp.inf), jnp.float32(0.0)) for _ in range(heads)
            )
            npair = (nwin + 1) // 2

            def pair_body(t, wc):
                eP, cbP, st = wc
                wait_rows(rowsA, semA)
                validB = 2 * t + 1 < nwin
                cbP = lax.cond(validB,
                               lambda: prep_issue(eP + 16, cbP, rowsB, semB),
                               lambda: cbP)
                st = compute_win(2 * t, rowsA, st, d)
                validA2 = 2 * t + 2 < nwin
                cbP = lax.cond(validA2,
                               lambda: prep_issue(eP + 32, cbP, rowsA, semA),
                               lambda: cbP)

                def do_b():
                    wait_rows(rowsB, semB)
                    return compute_win(2 * t + 1, rowsB, st, d)

                st = lax.cond(validB, do_b, lambda: st)
                return eP + 32, cbP, st

            _, cb, stF = lax.fori_loop(0, npair, pair_body, (e, cb, st0))

            for h in range(heads):
                _, den_h = stF[h]
                invv = 1.0 / (jnp.full((16,), den_h, jnp.float32) + 1e-16)

                def fq(q, _, h=h, invv=invv):
                    c0 = h * 256 + q * 16
                    o = accbuf[0, pl.ds(c0, 16)] * invv + bobuf[0, pl.ds(c0, 16)]
                    outbuf[0, pl.ds(c0, 16)] = jnp.maximum(o, 0.01 * o)
                    return 0

                lax.fori_loop(0, 16, fq, 0)

            pltpu.sync_copy(outbuf, out_hbm.at[pl.ds(n, 1)])
            return e + d, cb, db

        lax.fori_loop(
            n0, n1, node_body,
            (e0, jnp.int32(-(2 ** 30)), jnp.int32(-(2 ** 30))),
        )

    return k(xl, xr, srcp, degp, wtab, attf, bof)


def _final_ln(s4p, xf0p, batch_row, batch_col, lnw, lnb):
    def fk(s_ref, x0_ref, b_ref, bc_ref, w_ref, bb_ref, o_ref):
        h = s_ref[...] + x0_ref[...]
        bt = b_ref[...]
        gi = lax.broadcasted_iota(jnp.int32, (GP, NP), 0)
        oh = jnp.where(gi == bt, 1.0, 0.0).astype(jnp.float32)
        ni = lax.broadcasted_iota(jnp.int32, (GP, NP), 1)
        psel = jnp.where(ni == gi * 100, 1.0, 0.0).astype(jnp.float32)

        rs = jnp.sum(h, axis=1, keepdims=True)
        rq = jnp.sum(h * h, axis=1, keepdims=True)
        ones = jnp.ones((NP, 1), jnp.float32)
        stats = jnp.concatenate([rs, rq, ones], axis=1)
        g = jnp.dot(oh, stats, preferred_element_type=jnp.float32)
        cnt = g[:, 2:3]
        norm = jnp.clip(cnt, 1.0, None) * 256.0
        mean = g[:, 0:1] / norm
        var = g[:, 1:2] / norm - mean * mean
        rstd = lax.rsqrt(var + 1e-5)

        h_sel = jnp.dot(psel, h, preferred_element_type=jnp.float32)
        x0_sel = jnp.dot(psel, x0_ref[...], preferred_element_type=jnp.float32)
        b_sel = jnp.dot(psel, bc_ref[...],
                        preferred_element_type=jnp.float32)
        gi2 = lax.broadcasted_iota(jnp.int32, (GP, GP), 1)
        q = jnp.where(b_sel == gi2.astype(jnp.float32), 1.0, 0.0)
        mean_sel = jnp.dot(q, mean, preferred_element_type=jnp.float32)
        rstd_sel = jnp.dot(q, rstd, preferred_element_type=jnp.float32)

        o_ref[...] = ((h_sel - mean_sel) * rstd_sel * w_ref[...] + bb_ref[...]
                      + x0_sel)

    return pl.pallas_call(
        fk,
        out_shape=jax.ShapeDtypeStruct((GP, D), jnp.float32),
    )(s4p, xf0p, batch_row, batch_col, lnw[None, :], lnb[None, :])


def kernel(x, edge_index, batch, params):
    loops = jnp.arange(N, dtype=edge_index.dtype)
    src = jnp.concatenate([edge_index[0], loops])
    dst = jnp.concatenate([edge_index[1], loops])

    perm = jnp.argsort(dst)
    dst_s = dst[perm]
    src_s = src[perm].astype(jnp.int32)
    noff = jnp.searchsorted(dst_s, jnp.arange(N + 1)).astype(jnp.int32)
    deg = noff[1:] - noff[:-1]
    degp = jnp.pad(deg, (0, DEG_PAD - N))
    t = (jnp.arange(33) * E2) // 32
    nb = jnp.searchsorted(noff, t).astype(jnp.int32)
    eoff = noff[nb]
    wtab = jnp.zeros((WTAB,), jnp.int32)
    wtab = wtab.at[0:33].set(eoff).at[48:81].set(nb)
    srcp = jnp.pad(src_s, (0, SRC_PAD - E2))

    idxp = jnp.pad(x.reshape(-1).astype(jnp.int32), (0, 20480 - 2 * N))
    xf = _emb_gather(params["emb"], idxp)[: 2 * N].reshape(N, D)
    res = xf

    L = params["layers"]
    for i in range(5):
        p = L[i]
        heads = HEADS if i < 4 else 1
        xl = _matmul_bias(xf, p["Wl"], p["bl"])
        xr = _matmul_bias(xf, p["Wr"], p["br"])
        attf = p["att"].reshape(1, -1)
        bof = p["bo"][None, :]
        xf = _edge_layer(xl, xr, srcp, degp, wtab, attf, bof, heads)

    s4p = jnp.pad(xf, ((0, NP - N), (0, 0)))
    xf0p = jnp.pad(res, ((0, NP - N), (0, 0)))
    batchp = jnp.pad(batch.astype(jnp.int32), (0, NP - N),
                     constant_values=NUM_GRAPHS + 7)
    batch_row = batchp[None, :]
    batch_col = jnp.where(batchp < NUM_GRAPHS, batchp, 0).astype(jnp.float32)[:, None]
    out = _final_ln(s4p, xf0p, batch_row, batch_col,
                    params["ln_w"], params["ln_b"])
    return out[:NUM_GRAPHS]


# speedup vs baseline: 1.0258x; 1.0145x over previous
import dataclasses
import functools

import jax
import jax.numpy as jnp
from jax import lax
from jax.experimental import pallas as pl
from jax.experimental.pallas import tpu as pltpu
from jax.experimental.pallas import tpu_sc as plsc

FEAT = 128
HEADS = 4
MAX_GATE = 99
N_PER_GRAPH = MAX_GATE + 1
NUM_GRAPHS = 100
D = 2 * FEAT

N = 10000
E2 = 160000 + N
NW = 32
NP = 10240
GP = 104
SRC_PAD = E2 + 4096
DEG_PAD = N + 256
WTAB = 96


def _sc_params():
    cp = pltpu.CompilerParams()
    if "needs_layout_passes" in pltpu.CompilerParams.__dataclass_fields__:
        cp = dataclasses.replace(cp, needs_layout_passes=False)
    return cp


def _iota16():
    return lax.iota(jnp.int32, 16)


def _extract_i32(ref, pos, base, nchunks):
    acc = jnp.int32(0)
    for k in range(nchunks):
        v = ref[pl.ds(base + k * 16, 16)]
        acc = acc + jnp.sum(jnp.where(_iota16() + (base + k * 16) == pos, v, 0))
    return acc


def _extract_dyn_i32(ref, pos):
    g = (pos // 16) * 16
    v = ref[pl.ds(g, 16)]
    return jnp.sum(jnp.where(_iota16() == pos - g, v, 0))


def _emb_gather(emb, idxp):
    B = idxp.shape[0]
    b_per_w = B // NW
    mesh = plsc.VectorSubcoreMesh(core_axis_name="c", subcore_axis_name="s")

    @functools.partial(
        pl.kernel,
        out_type=jax.ShapeDtypeStruct((B, FEAT), jnp.float32),
        mesh=mesh,
        compiler_params=_sc_params(),
        scratch_types=[
            pltpu.VMEM((b_per_w,), jnp.int32),
            pltpu.VMEM((b_per_w, FEAT), jnp.float32),
            pltpu.SemaphoreType.DMA,
        ],
    )
    def k(emb_hbm, idx_hbm, out_hbm, idx_v, rows_v, sem):
        w = lax.axis_index("c") * 16 + lax.axis_index("s")
        base = w * b_per_w
        pltpu.sync_copy(idx_hbm.at[pl.ds(base, b_per_w)], idx_v)
        pltpu.async_copy(emb_hbm.at[idx_v], rows_v, sem).wait()
        pltpu.sync_copy(rows_v, out_hbm.at[pl.ds(base, b_per_w)])

    return k(emb, idxp)


def _matmul_bias(xp, w, b):
    Nr, K = xp.shape
    M = w.shape[1]
    BM, BN = 400, min(M, 512)
    grid = (Nr // BM, M // BN)

    def mk(x_ref, w_ref, b_ref, o_ref):
        o_ref[...] = (
            jnp.dot(x_ref[...].astype(jnp.bfloat16),
                    w_ref[...].astype(jnp.bfloat16),
                    preferred_element_type=jnp.float32)
            + b_ref[...]
        )

    return pl.pallas_call(
        mk,
        grid=grid,
        in_specs=[
            pl.BlockSpec((BM, K), lambda i, j: (i, 0)),
            pl.BlockSpec((K, BN), lambda i, j: (0, j)),
            pl.BlockSpec((1, BN), lambda i, j: (0, j)),
        ],
        out_specs=pl.BlockSpec((BM, BN), lambda i, j: (i, j)),
        out_shape=jax.ShapeDtypeStruct((Nr, M), jnp.float32),
    )(xp, w, b[None, :])


def _edge_layer(xl, xr, srcp, degp, wtab, attf, bof, heads):
    HC = heads * 256
    mesh = plsc.VectorSubcoreMesh(core_axis_name="c", subcore_axis_name="s")

    @functools.partial(
        pl.kernel,
        out_type=jax.ShapeDtypeStruct((N, HC), jnp.float32),
        mesh=mesh,
        compiler_params=_sc_params(),
        scratch_types=[
            pltpu.VMEM((2048,), jnp.int32),
            pltpu.VMEM((256,), jnp.int32),
            pltpu.VMEM((WTAB,), jnp.int32),
            pltpu.VMEM((16, HC), jnp.float32),
            pltpu.VMEM((16, HC), jnp.float32),
            pltpu.VMEM((1, HC), jnp.float32),
            pltpu.VMEM((1, HC), jnp.float32),
            pltpu.VMEM((1, HC), jnp.float32),
            pltpu.VMEM((1, HC), jnp.float32),
            pltpu.VMEM((1, HC), jnp.float32),
            pltpu.SemaphoreType.DMA,
            pltpu.SemaphoreType.DMA,
            pltpu.SemaphoreType.DMA,
            pltpu.SemaphoreType.DMA,
        ],
    )
    def k(xl_hbm, xr_hbm, src_hbm, deg_hbm, wtab_hbm, att_hbm, bo_hbm,
          out_hbm, srcbuf, degbuf, wtabbuf, rowsA, rowsB, xrbuf, accbuf,
          outbuf, attbuf, bobuf, sem, semA, semB, semO):
        w = lax.axis_index("c") * 16 + lax.axis_index("s")
        pltpu.sync_copy(wtab_hbm, wtabbuf)
        pltpu.sync_copy(att_hbm, attbuf)
        pltpu.sync_copy(bo_hbm, bobuf)
        e0 = _extract_i32(wtabbuf, w, 0, 3)
        n0 = _extract_i32(wtabbuf, 48 + w, 48, 3)
        n1 = _extract_i32(wtabbuf, 48 + w + 1, 48, 3)

        def prep_issue(epos, cb_, rowsX, semX):
            need_src = jnp.logical_or(epos + 16 > cb_ + 2048, cb_ < 0)

            def refill_src():
                sb_ = (epos // 16) * 16
                pltpu.sync_copy(src_hbm.at[pl.ds(sb_, 2048)], srcbuf)
                return sb_

            cb_ = lax.cond(need_src, refill_src, lambda: cb_)
            idxv = srcbuf[pl.ds(epos - cb_, 16)]
            pltpu.make_async_copy(xl_hbm.at[idxv], rowsX, semX).start()
            return cb_

        def wait_rows(rowsX, semX):
            dummy = jnp.zeros((16,), jnp.int32)
            pltpu.make_async_copy(xl_hbm.at[dummy], rowsX, semX).wait()

        def compute_win(wk, rowsX, st, d):
            drem = d - wk * 16
            lanemask = _iota16() < drem
            newst = []
            for h in range(heads):
                def aq(q, accs, h=h):
                    c0 = h * 256 + q * 16
                    xrv = xrbuf[0, pl.ds(c0, 16)]
                    atv = attbuf[0, pl.ds(c0, 16)]
                    out = []
                    for j in range(16):
                        z = rowsX[j, pl.ds(c0, 16)] + xrv
                        z = jnp.maximum(z, 0.2 * z)
                        out.append(accs[j] + z * atv)
                    return tuple(out)

                accs = lax.fori_loop(
                    0, 16, aq,
                    tuple(jnp.zeros((16,), jnp.float32) for _ in range(16)),
                )
                alph = jnp.full((16,), -jnp.inf, jnp.float32)
                for j in range(16):
                    alph = jnp.where(_iota16() == j, jnp.sum(accs[j]), alph)
                alph = jnp.where(lanemask, alph, -jnp.inf)
                m_old, den_old = st[h]
                m_new = jnp.maximum(m_old, jnp.max(alph))
                rv = jnp.exp(jnp.full((16,), m_old - m_new, jnp.float32))
                r_s = jnp.max(rv)
                wv = jnp.exp(alph - m_new)
                den_new = den_old * r_s + jnp.sum(wv)
                wsp = [wv[jnp.full((16,), j, jnp.int32)] for j in range(16)]

                def wq(q, _, h=h, wsp=wsp, r_s=r_s):
                    c0 = h * 256 + q * 16
                    a_ = accbuf[0, pl.ds(c0, 16)] * r_s
                    for j in range(16):
                        a_ = a_ + wsp[j] * rowsX[j, pl.ds(c0, 16)]
                    accbuf[0, pl.ds(c0, 16)] = a_
                    return 0

                lax.fori_loop(0, 16, wq, 0)
                newst.append((m_new, den_new))
            return tuple(newst)

        def node_body(n, carry):
            e, cb, db = carry

            need_deg = jnp.logical_or(n >= db + 256, db < 0)

            def refill_deg():
                nb_ = (n // 16) * 16
                pltpu.sync_copy(deg_hbm.at[pl.ds(nb_, 256)], degbuf)
                return nb_

            db = lax.cond(need_deg, refill_deg, lambda: db)
            d = _extract_dyn_i32(degbuf, n - db)

            xr_cp = pltpu.make_async_copy(xr_hbm.at[pl.ds(n, 1)], xrbuf, sem)
            xr_cp.start()

            nwin = (d + 15) // 16
            cb = lax.cond(nwin > 0,
                          lambda: prep_issue(e, cb, rowsA, semA),
                          lambda: cb)

            def zero_q(q, _):
                accbuf[0, pl.ds(q * 16, 16)] = jnp.zeros((16,), jnp.float32)
                return 0

            lax.fori_loop(0, HC // 16, zero_q, 0)
            xr_cp.wait()

            st0 = tuple(
                (jnp.float32(-jnp.inf), jnp.float32(0.0)) for _ in range(heads)
            )
            npair = (nwin + 1) // 2

            def pair_body(t, wc):
                eP, cbP, st = wc
                wait_rows(rowsA, semA)
                validB = 2 * t + 1 < nwin
                cbP = lax.cond(validB,
                               lambda: prep_issue(eP + 16, cbP, rowsB, semB),
                               lambda: cbP)
                st = compute_win(2 * t, rowsA, st, d)
                validA2 = 2 * t + 2 < nwin
                cbP = lax.cond(validA2,
                               lambda: prep_issue(eP + 32, cbP, rowsA, semA),
                               lambda: cbP)

                def do_b():
                    wait_rows(rowsB, semB)
                    return compute_win(2 * t + 1, rowsB, st, d)

                st = lax.cond(validB, do_b, lambda: st)
                return eP + 32, cbP, st

            _, cb, stF = lax.fori_loop(0, npair, pair_body, (e, cb, st0))

            @pl.when(n > n0)
            def _():
                pltpu.make_async_copy(out_hbm.at[pl.ds(0, 1)], outbuf,
                                      semO).wait()

            for h in range(heads):
                _, den_h = stF[h]
                invv = 1.0 / (jnp.full((16,), den_h, jnp.float32) + 1e-16)

                def fq(q, _, h=h, invv=invv):
                    c0 = h * 256 + q * 16
                    o = accbuf[0, pl.ds(c0, 16)] * invv + bobuf[0, pl.ds(c0, 16)]
                    outbuf[0, pl.ds(c0, 16)] = jnp.maximum(o, 0.01 * o)
                    return 0

                lax.fori_loop(0, 16, fq, 0)

            pltpu.make_async_copy(outbuf, out_hbm.at[pl.ds(n, 1)], semO).start()
            return e + d, cb, db

        lax.fori_loop(
            n0, n1, node_body,
            (e0, jnp.int32(-(2 ** 30)), jnp.int32(-(2 ** 30))),
        )

        @pl.when(n1 > n0)
        def _():
            pltpu.make_async_copy(out_hbm.at[pl.ds(0, 1)], outbuf, semO).wait()

    return k(xl, xr, srcp, degp, wtab, attf, bof)


def _final_ln(s4p, xf0p, batch_row, batch_col, lnw, lnb):
    def fk(s_ref, x0_ref, b_ref, bc_ref, w_ref, bb_ref, o_ref):
        h = s_ref[...] + x0_ref[...]
        bt = b_ref[...]
        gi = lax.broadcasted_iota(jnp.int32, (GP, NP), 0)
        oh = jnp.where(gi == bt, 1.0, 0.0).astype(jnp.float32)
        ni = lax.broadcasted_iota(jnp.int32, (GP, NP), 1)
        psel = jnp.where(ni == gi * 100, 1.0, 0.0).astype(jnp.float32)

        rs = jnp.sum(h, axis=1, keepdims=True)
        rq = jnp.sum(h * h, axis=1, keepdims=True)
        ones = jnp.ones((NP, 1), jnp.float32)
        stats = jnp.concatenate([rs, rq, ones], axis=1)
        g = jnp.dot(oh, stats, preferred_element_type=jnp.float32)
        cnt = g[:, 2:3]
        norm = jnp.clip(cnt, 1.0, None) * 256.0
        mean = g[:, 0:1] / norm
        var = g[:, 1:2] / norm - mean * mean
        rstd = lax.rsqrt(var + 1e-5)

        h_sel = jnp.dot(psel, h, preferred_element_type=jnp.float32)
        x0_sel = jnp.dot(psel, x0_ref[...], preferred_element_type=jnp.float32)
        b_sel = jnp.dot(psel, bc_ref[...],
                        preferred_element_type=jnp.float32)
        gi2 = lax.broadcasted_iota(jnp.int32, (GP, GP), 1)
        q = jnp.where(b_sel == gi2.astype(jnp.float32), 1.0, 0.0)
        mean_sel = jnp.dot(q, mean, preferred_element_type=jnp.float32)
        rstd_sel = jnp.dot(q, rstd, preferred_element_type=jnp.float32)

        o_ref[...] = ((h_sel - mean_sel) * rstd_sel * w_ref[...] + bb_ref[...]
                      + x0_sel)

    return pl.pallas_call(
        fk,
        out_shape=jax.ShapeDtypeStruct((GP, D), jnp.float32),
    )(s4p, xf0p, batch_row, batch_col, lnw[None, :], lnb[None, :])


def kernel(x, edge_index, batch, params):
    loops = jnp.arange(N, dtype=edge_index.dtype)
    src = jnp.concatenate([edge_index[0], loops])
    dst = jnp.concatenate([edge_index[1], loops])

    perm = jnp.argsort(dst)
    dst_s = dst[perm]
    src_s = src[perm].astype(jnp.int32)
    noff = jnp.searchsorted(dst_s, jnp.arange(N + 1)).astype(jnp.int32)
    deg = noff[1:] - noff[:-1]
    degp = jnp.pad(deg, (0, DEG_PAD - N))
    t = (jnp.arange(33) * E2) // 32
    nb = jnp.searchsorted(noff, t).astype(jnp.int32)
    eoff = noff[nb]
    wtab = jnp.zeros((WTAB,), jnp.int32)
    wtab = wtab.at[0:33].set(eoff).at[48:81].set(nb)
    srcp = jnp.pad(src_s, (0, SRC_PAD - E2))

    idxp = jnp.pad(x.reshape(-1).astype(jnp.int32), (0, 20480 - 2 * N))
    xf = _emb_gather(params["emb"], idxp)[: 2 * N].reshape(N, D)
    res = xf

    L = params["layers"]
    for i in range(5):
        p = L[i]
        heads = HEADS if i < 4 else 1
        xl = _matmul_bias(xf, p["Wl"], p["bl"])
        xr = _matmul_bias(xf, p["Wr"], p["br"])
        attf = p["att"].reshape(1, -1)
        bof = p["bo"][None, :]
        xf = _edge_layer(xl, xr, srcp, degp, wtab, attf, bof, heads)

    s4p = jnp.pad(xf, ((0, NP - N), (0, 0)))
    xf0p = jnp.pad(res, ((0, NP - N), (0, 0)))
    batchp = jnp.pad(batch.astype(jnp.int32), (0, NP - N),
                     constant_values=NUM_GRAPHS + 7)
    batch_row = batchp[None, :]
    batch_col = jnp.where(batchp < NUM_GRAPHS, batchp, 0).astype(jnp.float32)[:, None]
    out = _final_ln(s4p, xf0p, batch_row, batch_col,
                    params["ln_w"], params["ln_b"])
    return out[:NUM_GRAPHS]
